# Initial kernel scaffold; baseline (speedup 1.0000x reference)
import jax
import jax.numpy as jnp
from jax.experimental import pallas as pl


def kernel(idx, emb1, emb2, W1, b1, W2, b2, noise):
    raise NotImplementedError("write your pallas kernel here")


# speedup vs baseline: 10.9876x; 10.9876x over previous
import functools

import jax
import jax.numpy as jnp
from jax import lax
from jax.experimental import pallas as pl
from jax.experimental.pallas import tpu as pltpu
from jax.experimental.pallas import tpu_sc as plsc

ALPHA = 3.0
KTOP = 16


def _gather_sc(emb1, emb2, idx):
    B = idx.shape[0]
    D = emb1.shape[1]
    info = plsc.get_sparse_core_info()
    NC, NS = info.num_cores, info.num_subcores
    NW = NC * NS
    b_per_w = B // NW
    mesh = plsc.VectorSubcoreMesh(core_axis_name="c", subcore_axis_name="s")

    @functools.partial(
        pl.kernel,
        mesh=mesh,
        out_type=(
            jax.ShapeDtypeStruct((B, D), jnp.float32),
            jax.ShapeDtypeStruct((B, D), jnp.float32),
        ),
        scratch_types=[
            pltpu.VMEM((b_per_w,), jnp.int32),
            pltpu.VMEM((b_per_w, D), jnp.float32),
            pltpu.VMEM((b_per_w, D), jnp.float32),
            pltpu.SemaphoreType.DMA,
            pltpu.SemaphoreType.DMA,
        ],
    )
    def k(emb1_hbm, emb2_hbm, idx_hbm, out1_hbm, out2_hbm,
          idx_v, rows1_v, rows2_v, sem1, sem2):
        wid = lax.axis_index("s") * NC + lax.axis_index("c")
        base = wid * b_per_w
        pltpu.sync_copy(idx_hbm.at[pl.ds(base, b_per_w)], idx_v)
        c1 = pltpu.async_copy(emb1_hbm.at[idx_v], rows1_v, sem1)
        c2 = pltpu.async_copy(emb2_hbm.at[idx_v], rows2_v, sem2)
        c1.wait()
        pltpu.sync_copy(rows1_v, out1_hbm.at[pl.ds(base, b_per_w)])
        c2.wait()
        pltpu.sync_copy(rows2_v, out2_hbm.at[pl.ds(base, b_per_w)])

    return k(emb1, emb2, idx)


def _linear_body(g1_ref, g2_ref, w1_ref, b1_ref, w2_ref, b2_ref,
                 nv1_ref, nv2_ref):
    dn = (((1,), (1,)), ((), ()))
    z1 = lax.dot_general(g1_ref[...], w1_ref[...], dn,
                         preferred_element_type=jnp.float32)
    nv1_ref[...] = jnp.tanh(ALPHA * (z1 + b1_ref[...]))
    z2 = lax.dot_general(g2_ref[...], w2_ref[...], dn,
                         preferred_element_type=jnp.float32)
    nv2_ref[...] = jnp.tanh(ALPHA * (z2 + b2_ref[...]))


def _linear(g1, g2, W1, b1, W2, b2):
    B, D = g1.shape
    return pl.pallas_call(
        _linear_body,
        out_shape=(
            jax.ShapeDtypeStruct((B, D), jnp.float32),
            jax.ShapeDtypeStruct((B, D), jnp.float32),
        ),
    )(g1, g2, W1, b1.reshape(1, D), W2, b2.reshape(1, D))


def _adj_body(nv1_ref, nv2_ref, noise_ref, out_ref):
    i = pl.program_id(0)
    rows = noise_ref.shape[0]
    dn = (((1,), (1,)), ((), ()))
    nv1b = nv1_ref[pl.ds(i * rows, rows), :]
    nv2b = nv2_ref[pl.ds(i * rows, rows), :]
    m1 = lax.dot_general(nv1b, nv2_ref[...], dn,
                         preferred_element_type=jnp.float32)
    m2 = lax.dot_general(nv2b, nv1_ref[...], dn,
                         preferred_element_type=jnp.float32)
    adj = jnp.maximum(jnp.tanh(ALPHA * (m1 - m2)), 0.0)
    v = adj + noise_ref[...]
    work = v
    ncols = v.shape[1]
    col = lax.broadcasted_iota(jnp.int32, (rows, ncols), 1)
    for _ in range(KTOP):
        m = jnp.max(work, axis=1, keepdims=True)
        cand = jnp.where(work >= m, col, ncols)
        cmin = jnp.min(cand, axis=1, keepdims=True)
        work = jnp.where(col == cmin, -1.0, work)
    out_ref[...] = jnp.where(work < 0.0, adj, 0.0)


def _adj_topk(nv1, nv2, noise, block_rows=256):
    B = nv1.shape[0]
    nb = B // block_rows
    return pl.pallas_call(
        _adj_body,
        grid=(nb,),
        in_specs=[
            pl.BlockSpec((B, nv1.shape[1]), lambda i: (0, 0)),
            pl.BlockSpec((B, nv1.shape[1]), lambda i: (0, 0)),
            pl.BlockSpec((block_rows, B), lambda i: (i, 0)),
        ],
        out_specs=pl.BlockSpec((block_rows, B), lambda i: (i, 0)),
        out_shape=jax.ShapeDtypeStruct((B, B), jnp.float32),
    )(nv1, nv2, noise)


def kernel(idx, emb1, emb2, W1, b1, W2, b2, noise):
    g1, g2 = _gather_sc(emb1, emb2, idx)
    nv1, nv2 = _linear(g1, g2, W1, b1, W2, b2)
    return _adj_topk(nv1, nv2, noise)


# speedup vs baseline: 12.8377x; 1.1684x over previous
import functools

import jax
import jax.numpy as jnp
from jax import lax
from jax.experimental import pallas as pl
from jax.experimental.pallas import tpu as pltpu
from jax.experimental.pallas import tpu_sc as plsc

ALPHA = 3.0
KTOP = 16


def _gather_sc(emb1, emb2, idx):
    B = idx.shape[0]
    D = emb1.shape[1]
    info = plsc.get_sparse_core_info()
    NC, NS = info.num_cores, info.num_subcores
    NW = NC * NS
    b_per_w = B // NW
    mesh = plsc.VectorSubcoreMesh(core_axis_name="c", subcore_axis_name="s")

    @functools.partial(
        pl.kernel,
        mesh=mesh,
        out_type=(
            jax.ShapeDtypeStruct((B, D), jnp.float32),
            jax.ShapeDtypeStruct((B, D), jnp.float32),
        ),
        scratch_types=[
            pltpu.VMEM((b_per_w,), jnp.int32),
            pltpu.VMEM((b_per_w, D), jnp.float32),
            pltpu.VMEM((b_per_w, D), jnp.float32),
            pltpu.SemaphoreType.DMA,
            pltpu.SemaphoreType.DMA,
        ],
    )
    def k(emb1_hbm, emb2_hbm, idx_hbm, out1_hbm, out2_hbm,
          idx_v, rows1_v, rows2_v, sem1, sem2):
        wid = lax.axis_index("s") * NC + lax.axis_index("c")
        base = wid * b_per_w
        pltpu.sync_copy(idx_hbm.at[pl.ds(base, b_per_w)], idx_v)
        c1 = pltpu.async_copy(emb1_hbm.at[idx_v], rows1_v, sem1)
        c2 = pltpu.async_copy(emb2_hbm.at[idx_v], rows2_v, sem2)
        c1.wait()
        pltpu.sync_copy(rows1_v, out1_hbm.at[pl.ds(base, b_per_w)])
        c2.wait()
        pltpu.sync_copy(rows2_v, out2_hbm.at[pl.ds(base, b_per_w)])

    return k(emb1, emb2, idx)


def _linear_body(g1_ref, g2_ref, w1_ref, b1_ref, w2_ref, b2_ref,
                 nv1_ref, nv2_ref):
    dn = (((1,), (1,)), ((), ()))
    z1 = lax.dot_general(g1_ref[...], w1_ref[...], dn,
                         preferred_element_type=jnp.float32)
    nv1_ref[...] = jnp.tanh(ALPHA * (z1 + b1_ref[...]))
    z2 = lax.dot_general(g2_ref[...], w2_ref[...], dn,
                         preferred_element_type=jnp.float32)
    nv2_ref[...] = jnp.tanh(ALPHA * (z2 + b2_ref[...]))


def _linear(g1, g2, W1, b1, W2, b2):
    B, D = g1.shape
    return pl.pallas_call(
        _linear_body,
        out_shape=(
            jax.ShapeDtypeStruct((B, D), jnp.float32),
            jax.ShapeDtypeStruct((B, D), jnp.float32),
        ),
    )(g1, g2, W1, b1.reshape(1, D), W2, b2.reshape(1, D))


def _adj_body(nv1_ref, nv2_ref, noise_ref, out_ref):
    i = pl.program_id(0)
    rows = noise_ref.shape[0]
    dn = (((1,), (1,)), ((), ()))
    nv1b = nv1_ref[pl.ds(i * rows, rows), :]
    nv2b = nv2_ref[pl.ds(i * rows, rows), :]
    m1 = lax.dot_general(nv1b, nv2_ref[...], dn,
                         preferred_element_type=jnp.float32)
    m2 = lax.dot_general(nv2b, nv1_ref[...], dn,
                         preferred_element_type=jnp.float32)
    adj = jnp.maximum(jnp.tanh(ALPHA * (m1 - m2)), 0.0)
    v = adj + noise_ref[...]
    work = v
    ncols = v.shape[1]
    colf = lax.broadcasted_iota(jnp.int32, (rows, ncols), 1).astype(jnp.float32)
    for _ in range(KTOP):
        m = jnp.max(work, axis=1, keepdims=True)
        cand = jnp.where(work >= m, colf, float(ncols))
        cmin = jnp.min(cand, axis=1, keepdims=True)
        work = jnp.where(cand <= cmin, -1.0, work)
    out_ref[...] = jnp.where(work < 0.0, adj, 0.0)


def _adj_topk(nv1, nv2, noise, block_rows=256):
    B = nv1.shape[0]
    nb = B // block_rows
    return pl.pallas_call(
        _adj_body,
        grid=(nb,),
        in_specs=[
            pl.BlockSpec((B, nv1.shape[1]), lambda i: (0, 0)),
            pl.BlockSpec((B, nv1.shape[1]), lambda i: (0, 0)),
            pl.BlockSpec((block_rows, B), lambda i: (i, 0)),
        ],
        out_specs=pl.BlockSpec((block_rows, B), lambda i: (i, 0)),
        out_shape=jax.ShapeDtypeStruct((B, B), jnp.float32),
    )(nv1, nv2, noise)


def kernel(idx, emb1, emb2, W1, b1, W2, b2, noise):
    g1, g2 = _gather_sc(emb1, emb2, idx)
    nv1, nv2 = _linear(g1, g2, W1, b1, W2, b2)
    return _adj_topk(nv1, nv2, noise)


# speedup vs baseline: 13.1330x; 1.0230x over previous
import functools

import jax
import jax.numpy as jnp
from jax import lax
from jax.experimental import pallas as pl
from jax.experimental.pallas import tpu as pltpu
from jax.experimental.pallas import tpu_sc as plsc

ALPHA = 3.0
KTOP = 16


def _gather_sc(emb1, emb2, idx):
    B = idx.shape[0]
    D = emb1.shape[1]
    info = plsc.get_sparse_core_info()
    NC, NS = info.num_cores, info.num_subcores
    NW = NC * NS
    b_per_w = B // NW
    mesh = plsc.VectorSubcoreMesh(core_axis_name="c", subcore_axis_name="s")

    @functools.partial(
        pl.kernel,
        mesh=mesh,
        out_type=(
            jax.ShapeDtypeStruct((B, D), jnp.float32),
            jax.ShapeDtypeStruct((B, D), jnp.float32),
        ),
        scratch_types=[
            pltpu.VMEM((b_per_w,), jnp.int32),
            pltpu.VMEM((b_per_w, D), jnp.float32),
            pltpu.VMEM((b_per_w, D), jnp.float32),
            pltpu.SemaphoreType.DMA,
            pltpu.SemaphoreType.DMA,
        ],
    )
    def k(emb1_hbm, emb2_hbm, idx_hbm, out1_hbm, out2_hbm,
          idx_v, rows1_v, rows2_v, sem1, sem2):
        wid = lax.axis_index("s") * NC + lax.axis_index("c")
        base = wid * b_per_w
        pltpu.sync_copy(idx_hbm.at[pl.ds(base, b_per_w)], idx_v)
        c1 = pltpu.async_copy(emb1_hbm.at[idx_v], rows1_v, sem1)
        c2 = pltpu.async_copy(emb2_hbm.at[idx_v], rows2_v, sem2)
        c1.wait()
        pltpu.sync_copy(rows1_v, out1_hbm.at[pl.ds(base, b_per_w)])
        c2.wait()
        pltpu.sync_copy(rows2_v, out2_hbm.at[pl.ds(base, b_per_w)])

    return k(emb1, emb2, idx)


def _adj_body(g1_ref, g2_ref, w1_ref, b1_ref, w2_ref, b2_ref, noise_ref,
              out_ref, nv1_s, nv2_s):
    i = pl.program_id(0)
    dn = (((1,), (1,)), ((), ()))

    @pl.when(i == 0)
    def _():
        z1 = lax.dot_general(g1_ref[...], w1_ref[...], dn,
                             preferred_element_type=jnp.float32)
        nv1_s[...] = jnp.tanh(ALPHA * (z1 + b1_ref[...]))
        z2 = lax.dot_general(g2_ref[...], w2_ref[...], dn,
                             preferred_element_type=jnp.float32)
        nv2_s[...] = jnp.tanh(ALPHA * (z2 + b2_ref[...]))

    rows = noise_ref.shape[0]
    nv1b = nv1_s[pl.ds(i * rows, rows), :]
    nv2b = nv2_s[pl.ds(i * rows, rows), :]
    m1 = lax.dot_general(nv1b, nv2_s[...], dn,
                         preferred_element_type=jnp.float32)
    m2 = lax.dot_general(nv2b, nv1_s[...], dn,
                         preferred_element_type=jnp.float32)
    adj = jnp.maximum(jnp.tanh(ALPHA * (m1 - m2)), 0.0)
    v = adj + noise_ref[...]
    work = v
    ncols = v.shape[1]
    colf = lax.broadcasted_iota(jnp.int32, (rows, ncols), 1).astype(jnp.float32)
    for _ in range(KTOP):
        m = jnp.max(work, axis=1, keepdims=True)
        cand = jnp.where(work >= m, colf, float(ncols))
        cmin = jnp.min(cand, axis=1, keepdims=True)
        work = jnp.where(cand <= cmin, -1.0, work)
    out_ref[...] = jnp.where(work < 0.0, adj, 0.0)


def _adj_topk(g1, g2, W1, b1, W2, b2, noise, block_rows=256):
    B, D = g1.shape
    nb = noise.shape[0] // block_rows
    const = lambda i: (0, 0)
    return pl.pallas_call(
        _adj_body,
        grid=(nb,),
        in_specs=[
            pl.BlockSpec((B, D), const),
            pl.BlockSpec((B, D), const),
            pl.BlockSpec((D, D), const),
            pl.BlockSpec((1, D), const),
            pl.BlockSpec((D, D), const),
            pl.BlockSpec((1, D), const),
            pl.BlockSpec((block_rows, noise.shape[1]), lambda i: (i, 0)),
        ],
        out_specs=pl.BlockSpec((block_rows, noise.shape[1]), lambda i: (i, 0)),
        out_shape=jax.ShapeDtypeStruct(noise.shape, jnp.float32),
        scratch_shapes=[
            pltpu.VMEM((B, D), jnp.float32),
            pltpu.VMEM((B, D), jnp.float32),
        ],
    )(g1, g2, W1, b1.reshape(1, D), W2, b2.reshape(1, D), noise)


def kernel(idx, emb1, emb2, W1, b1, W2, b2, noise):
    g1, g2 = _gather_sc(emb1, emb2, idx)
    return _adj_topk(g1, g2, W1, b1, W2, b2, noise)
